# baseline (speedup 1.0000x reference)
import functools

import jax
import jax.numpy as jnp
from jax import lax
from jax.experimental import pallas as pl
from jax.experimental.pallas import tpu as pltpu
from jax.experimental.pallas import tpu_sc as plsc

EMBED_DIM = 64
SUB = 128
NSUB = 4
CHUNK = SUB * NSUB


def _make_gather(num_rows: int, nw: int):
    b_per_w = num_rows // nw
    n_iter = b_per_w // CHUNK
    assert b_per_w % CHUNK == 0

    mesh = plsc.VectorSubcoreMesh(core_axis_name="c", subcore_axis_name="s")

    @functools.partial(
        pl.kernel,
        mesh=mesh,
        out_type=jax.ShapeDtypeStruct((num_rows, EMBED_DIM), jnp.float32),
        scratch_types=[
            pltpu.VMEM((NSUB, SUB), jnp.int32),
            pltpu.VMEM((CHUNK, EMBED_DIM), jnp.float32),
            pltpu.SemaphoreType.DMA,
        ],
        compiler_params=pltpu.CompilerParams(use_tc_tiling_on_sc=False),
    )
    def k(table_hbm, idx_hbm, out_hbm, idx_v, rows_v, sem):
        nc = 2
        wid = lax.axis_index("s") * nc + lax.axis_index("c")
        base_row = wid * (b_per_w // SUB)

        def body(g, _):
            crow = base_row + g * NSUB
            pltpu.sync_copy(idx_hbm.at[pl.ds(crow, NSUB)], idx_v)
            copies = []
            for j in range(NSUB):
                copies.append(
                    pltpu.async_copy(
                        table_hbm.at[idx_v.at[j]],
                        rows_v.at[pl.ds(j * SUB, SUB)],
                        sem,
                    )
                )
            for c in copies:
                c.wait()
            pltpu.sync_copy(
                rows_v, out_hbm.at[pl.ds(wid * b_per_w + g * CHUNK, CHUNK)]
            )
            return 0

        lax.fori_loop(0, n_iter, body, 0)

    return k


def kernel(table, input):
    rows, cols = input.shape
    num_rows = rows * cols
    idx = input.reshape(num_rows // SUB, SUB).astype(jnp.int32)
    out = _make_gather(num_rows, 32)(table, idx)
    return out.reshape(rows, cols, EMBED_DIM)


# speedup vs baseline: 1.0551x; 1.0551x over previous
import functools

import jax
import jax.numpy as jnp
from jax import lax
from jax.experimental import pallas as pl
from jax.experimental.pallas import tpu as pltpu
from jax.experimental.pallas import tpu_sc as plsc

EMBED_DIM = 64
SUB = 128
NSUB = 4
CHUNK = SUB * NSUB


def _make_gather(num_rows: int, nw: int):
    b_per_w = num_rows // nw
    n_iter = b_per_w // CHUNK
    assert b_per_w % CHUNK == 0 and n_iter % 2 == 0

    mesh = plsc.VectorSubcoreMesh(core_axis_name="c", subcore_axis_name="s")

    @functools.partial(
        pl.kernel,
        mesh=mesh,
        out_type=jax.ShapeDtypeStruct((num_rows, EMBED_DIM), jnp.float32),
        scratch_types=[
            pltpu.VMEM((2, NSUB, SUB), jnp.int32),
            pltpu.VMEM((2, CHUNK, EMBED_DIM), jnp.float32),
            pltpu.SemaphoreType.DMA,
            pltpu.SemaphoreType.DMA,
            pltpu.SemaphoreType.DMA,
            pltpu.SemaphoreType.DMA,
        ],
        compiler_params=pltpu.CompilerParams(use_tc_tiling_on_sc=False),
    )
    def k(table_hbm, idx_hbm, out_hbm, idx_v, rows_v, sg0, sg1, so0, so1):
        nc = 2
        wid = lax.axis_index("s") * nc + lax.axis_index("c")
        ibase = wid * (b_per_w // SUB)
        obase = wid * b_per_w
        sg = (sg0, sg1)
        so = (so0, so1)

        def fire(g, slot):
            pltpu.sync_copy(idx_hbm.at[pl.ds(ibase + g * NSUB, NSUB)],
                            idx_v.at[slot])
            for j in range(NSUB):
                pltpu.async_copy(
                    table_hbm.at[idx_v.at[slot, j]],
                    rows_v.at[slot, pl.ds(j * SUB, SUB)],
                    sg[slot],
                )

        def drain(slot):
            for j in range(NSUB):
                pltpu.make_async_copy(
                    table_hbm.at[idx_v.at[slot, j]],
                    rows_v.at[slot, pl.ds(j * SUB, SUB)],
                    sg[slot],
                ).wait()

        def write_out(g, slot):
            pltpu.async_copy(rows_v.at[slot],
                             out_hbm.at[pl.ds(obase + g * CHUNK, CHUNK)],
                             so[slot])

        def wait_write(g, slot):
            pltpu.make_async_copy(rows_v.at[slot],
                                  out_hbm.at[pl.ds(obase + g * CHUNK, CHUNK)],
                                  so[slot]).wait()

        fire(0, 0)

        def body(i, _):
            for slot in (0, 1):
                g = 2 * i + slot
                other = 1 - slot
                if slot == 0:
                    @pl.when(g >= 1)
                    def _():
                        wait_write(g - 1, other)

                    fire(g + 1, other)
                else:
                    wait_write(g - 1, other)

                    @pl.when(g + 1 < n_iter)
                    def _():
                        fire(g + 1, other)
                drain(slot)
                write_out(g, slot)
            return 0

        lax.fori_loop(0, n_iter // 2, body, 0)
        wait_write(n_iter - 1, 1)

    return k


def kernel(table, input):
    rows, cols = input.shape
    num_rows = rows * cols
    idx = input.reshape(num_rows // SUB, SUB).astype(jnp.int32)
    out = _make_gather(num_rows, 32)(table, idx)
    return out.reshape(rows, cols, EMBED_DIM)


# speedup vs baseline: 1.0555x; 1.0004x over previous
import functools

import jax
import jax.numpy as jnp
from jax import lax
from jax.experimental import pallas as pl
from jax.experimental.pallas import tpu as pltpu
from jax.experimental.pallas import tpu_sc as plsc

EMBED_DIM = 64
SEQ = 200
SUB = 100
NS_C = 4
NSUB = NS_C * (SEQ // SUB)
CHUNK = NS_C * SEQ


def _make_gather(n_sent: int, nw: int):
    sent_per_w = n_sent // nw
    n_iter = sent_per_w // NS_C
    assert sent_per_w % NS_C == 0 and n_iter % 2 == 0

    mesh = plsc.VectorSubcoreMesh(core_axis_name="c", subcore_axis_name="s")

    @functools.partial(
        pl.kernel,
        mesh=mesh,
        out_type=jax.ShapeDtypeStruct((n_sent, SEQ, EMBED_DIM), jnp.float32),
        scratch_types=[
            pltpu.VMEM((2, NSUB, SUB), jnp.int32),
            pltpu.VMEM((2, NS_C, SEQ, EMBED_DIM), jnp.float32),
            pltpu.SemaphoreType.DMA,
            pltpu.SemaphoreType.DMA,
            pltpu.SemaphoreType.DMA,
            pltpu.SemaphoreType.DMA,
        ],
        compiler_params=pltpu.CompilerParams(use_tc_tiling_on_sc=False),
    )
    def k(table_hbm, idx_hbm, out_hbm, idx_v, rows_v, sg0, sg1, so0, so1):
        nc = 2
        wid = lax.axis_index("s") * nc + lax.axis_index("c")
        ibase = wid * (sent_per_w * SEQ // SUB)
        obase = wid * sent_per_w
        sg = (sg0, sg1)
        so = (so0, so1)

        def fire(g, slot):
            pltpu.sync_copy(idx_hbm.at[pl.ds(ibase + g * NSUB, NSUB)],
                            idx_v.at[slot])
            for j in range(NSUB):
                pltpu.async_copy(
                    table_hbm.at[idx_v.at[slot, j]],
                    rows_v.at[slot, j // 2, pl.ds((j % 2) * SUB, SUB)],
                    sg[slot],
                )

        def drain(slot):
            for j in range(NSUB):
                pltpu.make_async_copy(
                    table_hbm.at[idx_v.at[slot, j]],
                    rows_v.at[slot, j // 2, pl.ds((j % 2) * SUB, SUB)],
                    sg[slot],
                ).wait()

        def write_out(g, slot):
            pltpu.async_copy(rows_v.at[slot],
                             out_hbm.at[pl.ds(obase + g * NS_C, NS_C)],
                             so[slot])

        def wait_write(g, slot):
            pltpu.make_async_copy(rows_v.at[slot],
                                  out_hbm.at[pl.ds(obase + g * NS_C, NS_C)],
                                  so[slot]).wait()

        fire(0, 0)

        def body(i, _):
            for slot in (0, 1):
                g = 2 * i + slot
                other = 1 - slot
                if slot == 0:
                    @pl.when(g >= 1)
                    def _():
                        wait_write(g - 1, other)

                    fire(g + 1, other)
                else:
                    wait_write(g - 1, other)

                    @pl.when(g + 1 < n_iter)
                    def _():
                        fire(g + 1, other)
                drain(slot)
                write_out(g, slot)
            return 0

        lax.fori_loop(0, n_iter // 2, body, 0)
        wait_write(n_iter - 1, 1)

    return k


def kernel(table, input):
    n_sent, seq = input.shape
    idx = input.reshape(n_sent * seq // SUB, SUB).astype(jnp.int32)
    return _make_gather(n_sent, 32)(table, idx)
